# baseline (speedup 1.0000x reference)
import functools

import jax
import jax.numpy as jnp
from jax import lax
from jax.experimental import pallas as pl
from jax.experimental.pallas import tpu as pltpu
from jax.experimental.pallas import tpu_sc as plsc

_NUM_CORES = 2
_NUM_SUBCORES = 16
_NUM_WORKERS = _NUM_CORES * _NUM_SUBCORES
_CHUNK = 104
_VB = 16640


@functools.lru_cache(maxsize=None)
def _make_tc_merge(d_freeze, d_train, vocab):
    d_out = d_freeze + d_train
    grid = (vocab + _VB - 1) // _VB

    def body(f_ref, t_ref, o_ref):
        f = jnp.transpose(f_ref[...], (1, 0))
        t = jnp.transpose(t_ref[...], (1, 0))
        o_ref[...] = jnp.concatenate([f, t], axis=1)

    return pl.pallas_call(
        body,
        grid=(grid,),
        in_specs=[
            pl.BlockSpec((d_freeze, _VB), lambda i: (0, i)),
            pl.BlockSpec((d_train, _VB), lambda i: (0, i)),
        ],
        out_specs=pl.BlockSpec((_VB, d_out), lambda i: (i, 0)),
        out_shape=jax.ShapeDtypeStruct((vocab, d_out), jnp.float32),
    )


@functools.lru_cache(maxsize=None)
def _make_sc_gather(n_idx, d_out):
    bpw = n_idx // _NUM_WORKERS
    n_chunks = bpw // _CHUNK
    nbuf = 8
    assert n_idx % _NUM_WORKERS == 0 and bpw % _CHUNK == 0
    assert n_chunks % nbuf == 0

    mesh = plsc.VectorSubcoreMesh(
        core_axis_name="c", subcore_axis_name="s",
        num_cores=_NUM_CORES, num_subcores=_NUM_SUBCORES)

    @functools.partial(
        pl.kernel,
        out_type=jax.ShapeDtypeStruct((n_idx, d_out), jnp.float32),
        mesh=mesh,
        scratch_types=[
            pltpu.VMEM((bpw,), jnp.int32),
            pltpu.VMEM((nbuf, _CHUNK, d_out), jnp.float32),
            [pltpu.SemaphoreType.DMA] * nbuf,
            [pltpu.SemaphoreType.DMA] * nbuf,
        ],
    )
    def sc_gather(table_hbm, idx_hbm, out_hbm, idx_v, rows, gsems, wsems):
        wid = lax.axis_index("s") * _NUM_CORES + lax.axis_index("c")
        base = wid * bpw
        pltpu.sync_copy(idx_hbm.at[pl.ds(base, bpw)], idx_v)

        def drain_write(b):
            pltpu.make_async_copy(
                rows.at[b], out_hbm.at[pl.ds(base, _CHUNK)], wsems[b]).wait()

        def group_body(j, carry):
            c0 = j * nbuf
            for b in range(nbuf):

                @pl.when(j > 0)
                def _():
                    drain_write(b)

                off = (c0 + b) * _CHUNK
                pltpu.async_copy(
                    table_hbm.at[idx_v.at[pl.ds(off, _CHUNK)]],
                    rows.at[b], gsems[b])
            for b in range(nbuf):
                pltpu.make_async_copy(
                    table_hbm.at[pl.ds(0, _CHUNK)], rows.at[b], gsems[b]).wait()
                row0 = base + (c0 + b) * _CHUNK
                pltpu.async_copy(rows.at[b],
                                 out_hbm.at[pl.ds(row0, _CHUNK)], wsems[b])
            return carry

        lax.fori_loop(0, n_chunks // nbuf, group_body, 0)
        for b in range(nbuf):
            drain_write(b)

    return sc_gather


def kernel(input_ids, freeze_buffer, training_part):
    b, s = input_ids.shape
    n_idx = b * s
    vocab, d_freeze = freeze_buffer.shape
    d_train = training_part.shape[1]
    d_out = d_freeze + d_train

    merged = _make_tc_merge(d_freeze, d_train, vocab)(
        jnp.transpose(freeze_buffer, (1, 0)), jnp.transpose(training_part, (1, 0)))
    idx = jnp.reshape(jnp.transpose(input_ids, (1, 0)), (n_idx,)).astype(jnp.int32)
    out = _make_sc_gather(n_idx, d_out)(merged, idx)
    return jnp.transpose(jnp.reshape(out, (s, b, d_out)), (1, 0, 2))


# speedup vs baseline: 1.0400x; 1.0400x over previous
import functools

import jax
import jax.numpy as jnp
from jax import lax
from jax.experimental import pallas as pl
from jax.experimental.pallas import tpu as pltpu
from jax.experimental.pallas import tpu_sc as plsc

_NUM_CORES = 2
_NUM_SUBCORES = 16
_NUM_WORKERS = _NUM_CORES * _NUM_SUBCORES
_CHUNK = 104
_VB = 12800


@functools.lru_cache(maxsize=None)
def _make_tc_merge(d_freeze, d_train, vocab):
    d_out = d_freeze + d_train
    grid = (vocab + _VB - 1) // _VB

    def body(f_ref, t_ref, o_ref):
        f = jnp.transpose(f_ref[...], (1, 0))
        t = jnp.transpose(t_ref[...], (1, 0))
        o_ref[...] = jnp.concatenate([f, t], axis=1)

    return pl.pallas_call(
        body,
        grid=(grid,),
        in_specs=[
            pl.BlockSpec((d_freeze, _VB), lambda i: (0, i)),
            pl.BlockSpec((d_train, _VB), lambda i: (0, i)),
        ],
        out_specs=pl.BlockSpec((_VB, d_out), lambda i: (i, 0)),
        out_shape=jax.ShapeDtypeStruct((vocab, d_out), jnp.float32),
    )


@functools.lru_cache(maxsize=None)
def _make_sc_gather(n_idx, d_out):
    bpw = n_idx // _NUM_WORKERS
    n_chunks = bpw // _CHUNK
    nbuf = 8
    assert n_idx % _NUM_WORKERS == 0 and bpw % _CHUNK == 0
    assert n_chunks % nbuf == 0

    mesh = plsc.VectorSubcoreMesh(
        core_axis_name="c", subcore_axis_name="s",
        num_cores=_NUM_CORES, num_subcores=_NUM_SUBCORES)

    @functools.partial(
        pl.kernel,
        out_type=jax.ShapeDtypeStruct((n_idx, d_out), jnp.float32),
        mesh=mesh,
        scratch_types=[
            pltpu.VMEM((bpw,), jnp.int32),
            pltpu.VMEM((nbuf, _CHUNK, d_out), jnp.float32),
            [pltpu.SemaphoreType.DMA] * nbuf,
            [pltpu.SemaphoreType.DMA] * nbuf,
        ],
    )
    def sc_gather(table_hbm, idx_hbm, out_hbm, idx_v, rows, gsems, wsems):
        wid = lax.axis_index("s") * _NUM_CORES + lax.axis_index("c")
        base = wid * bpw
        pltpu.sync_copy(idx_hbm.at[pl.ds(base, bpw)], idx_v)

        def drain_write(b):
            pltpu.make_async_copy(
                rows.at[b], out_hbm.at[pl.ds(base, _CHUNK)], wsems[b]).wait()

        def group_body(j, carry):
            c0 = j * nbuf
            for b in range(nbuf):

                @pl.when(j > 0)
                def _():
                    drain_write(b)

                off = (c0 + b) * _CHUNK
                pltpu.async_copy(
                    table_hbm.at[idx_v.at[pl.ds(off, _CHUNK)]],
                    rows.at[b], gsems[b])
            for b in range(nbuf):
                pltpu.make_async_copy(
                    table_hbm.at[pl.ds(0, _CHUNK)], rows.at[b], gsems[b]).wait()
                row0 = base + (c0 + b) * _CHUNK
                pltpu.async_copy(rows.at[b],
                                 out_hbm.at[pl.ds(row0, _CHUNK)], wsems[b])
            return carry

        lax.fori_loop(0, n_chunks // nbuf, group_body, 0)
        for b in range(nbuf):
            drain_write(b)

    return sc_gather


def kernel(input_ids, freeze_buffer, training_part):
    b, s = input_ids.shape
    n_idx = b * s
    vocab, d_freeze = freeze_buffer.shape
    d_train = training_part.shape[1]
    d_out = d_freeze + d_train

    merged = _make_tc_merge(d_freeze, d_train, vocab)(
        jnp.transpose(freeze_buffer, (1, 0)), jnp.transpose(training_part, (1, 0)))
    idx = jnp.reshape(jnp.transpose(input_ids, (1, 0)), (n_idx,)).astype(jnp.int32)
    out = _make_sc_gather(n_idx, d_out)(merged, idx)
    return jnp.transpose(jnp.reshape(out, (s, b, d_out)), (1, 0, 2))
